# baseline (speedup 1.0000x reference)
import jax
import jax.numpy as jnp
from jax import lax
from jax.experimental import pallas as pl
from jax.experimental.pallas import tpu as pltpu

_N, _T, _M, _K = 4096, 200, 16, 4
_LAN = _M * _K
_BN = 64
_BR = _BN * _T


def _stage1_body(mu_ref, x_ref, mask_ref, out_ref):
    a4 = lax.broadcasted_iota(jnp.int32, (_K, _LAN), 0)
    l4 = lax.broadcasted_iota(jnp.int32, (_K, _LAN), 1)
    p4 = jnp.where(a4 == l4 % _K, 1.0, 0.0).astype(jnp.float32)
    xe = lax.dot_general(x_ref[...], p4, (((1,), (0,)), ((), ())),
                         preferred_element_type=jnp.float32)
    d = mu_ref[...] - xe
    ad = jnp.abs(d)
    mn = jnp.minimum(ad, 1.0)
    h2 = mn * (2.0 * ad - mn)
    hm = h2 * mask_ref[...]
    s = jnp.sum(hm.reshape(_BN, _T, _LAN), axis=1)
    mrow = lax.broadcasted_iota(jnp.int32, (_LAN, _M), 0)
    mcol = lax.broadcasted_iota(jnp.int32, (_LAN, _M), 1)
    r = jnp.where(mrow // _K == mcol, 0.5, 0.0).astype(jnp.float32)
    out_ref[...] = lax.dot_general(s, r, (((1,), (0,)), ((), ())),
                                   preferred_element_type=jnp.float32)


def _stage2_body(mt_ref, out_ref):
    v = mt_ref[...]
    mn1 = jnp.min(v, axis=1, keepdims=True)
    gt = jnp.where(v > mn1, v, jnp.float32(jnp.inf))
    mn2 = jnp.min(gt, axis=1, keepdims=True)
    cnt = jnp.sum(jnp.where(v == mn1, 1.0, 0.0).astype(jnp.float32),
                  axis=1, keepdims=True)
    second = jnp.where(cnt > 1.5, mn1, mn2)
    out_ref[...] = jnp.sum(mn1 + second).reshape(1, 1)


def kernel(mu, x, mask, w):
    mu2 = mu.reshape(_N * _T, _LAN)
    x2 = x.reshape(_N * _T, _K)
    mask2 = mask.reshape(_N * _T, 1)
    mt = pl.pallas_call(
        _stage1_body,
        grid=(_N // _BN,),
        in_specs=[
            pl.BlockSpec((_BR, _LAN), lambda i: (i, 0)),
            pl.BlockSpec((_BR, _K), lambda i: (i, 0)),
            pl.BlockSpec((_BR, 1), lambda i: (i, 0)),
        ],
        out_specs=pl.BlockSpec((_BN, _M), lambda i: (i, 0)),
        out_shape=jax.ShapeDtypeStruct((_N, _M), jnp.float32),
        compiler_params=pltpu.CompilerParams(
            dimension_semantics=("parallel",)),
    )(mu2, x2, mask2)
    total = pl.pallas_call(
        _stage2_body,
        out_shape=jax.ShapeDtypeStruct((1, 1), jnp.float32),
    )(mt)
    return total[0, 0] / (_N * w)
